# Initial kernel scaffold; baseline (speedup 1.0000x reference)
import jax
import jax.numpy as jnp
from jax.experimental import pallas as pl


def kernel(x, edge_index, W1_self, W1_neigh, b1, W2_self, W2_neigh, b2):
    raise NotImplementedError("write your pallas kernel here")


# speedup vs baseline: 5.3346x; 5.3346x over previous
import functools

import jax
import jax.numpy as jnp
from jax import lax
from jax.experimental import pallas as pl
from jax.experimental.pallas import tpu as pltpu
from jax.experimental.pallas import tpu_sc as plsc

N = 10000
E = 320000
D = 128

NC = 2
NS = 16
NW = NC * NS
CH = 80
EW = E // NW
NCHUNK = EW // CH
W0 = 624
WROWS = 640
ZCH = 128
BN = 1000


def _make_sc_agg(with_deg):
  mesh = plsc.VectorSubcoreMesh(core_axis_name="c", subcore_axis_name="s")
  out_type = [jax.ShapeDtypeStruct((NC, N, D), jnp.float32)]
  scratch = [
      pltpu.VMEM((CH,), jnp.int32),
      pltpu.VMEM((CH,), jnp.int32),
      pltpu.VMEM((CH, D), jnp.float32),
      pltpu.VMEM((ZCH, D), jnp.float32),
      pltpu.VMEM_SHARED((N, D), jnp.float32),
      pltpu.SemaphoreType.DMA,
  ]
  if with_deg:
    out_type.append(jax.ShapeDtypeStruct((NC, N, 16), jnp.float32))
    scratch += [
        pltpu.VMEM((CH, 16), jnp.float32),
        pltpu.VMEM((WROWS, 16), jnp.float32),
        pltpu.VMEM_SHARED((N, 16), jnp.float32),
    ]

  def body(*refs):
    if with_deg:
      (xw, src, dst, zrow, zdeg, ones_h,
       agg_out, deg_out,
       sidx, didx, rows, zv, agg_sh, sem, ones_v, zdeg_v, deg_sh) = refs
    else:
      (xw, src, dst, zrow,
       agg_out,
       sidx, didx, rows, zv, agg_sh, sem) = refs
    c = lax.axis_index("c")
    s = lax.axis_index("s")
    wid = s * NC + c
    row0 = s * W0

    pltpu.sync_copy(zrow, zv)
    for k in range(WROWS // ZCH):
      pltpu.sync_copy(zv, agg_sh.at[pl.ds(row0 + k * ZCH, ZCH)])
    if with_deg:
      pltpu.sync_copy(ones_h, ones_v)
      pltpu.sync_copy(zdeg, zdeg_v)
      pltpu.sync_copy(zdeg_v, deg_sh.at[pl.ds(row0, WROWS)])
    plsc.subcore_barrier()

    def chunk(j, carry):
      base = wid * EW + j * CH
      pltpu.sync_copy(src.at[pl.ds(base, CH)], sidx)
      pltpu.sync_copy(dst.at[pl.ds(base, CH)], didx)
      pltpu.async_copy(xw.at[sidx], rows, sem).wait()
      pltpu.sync_copy(rows, agg_sh.at[didx], add=True)
      if with_deg:
        pltpu.sync_copy(ones_v, deg_sh.at[didx], add=True)
      return carry

    lax.fori_loop(0, NCHUNK, chunk, 0)
    plsc.subcore_barrier()

    pltpu.sync_copy(agg_sh.at[pl.ds(row0, WROWS)],
                    agg_out.at[c, pl.ds(row0, WROWS)])
    if with_deg:
      pltpu.sync_copy(deg_sh.at[pl.ds(row0, WROWS)],
                      deg_out.at[c, pl.ds(row0, WROWS)])

  return pl.kernel(
      body, out_type=out_type, mesh=mesh, scratch_types=scratch,
      compiler_params=pltpu.CompilerParams(use_tc_tiling_on_sc=False))


_sc_agg_deg = _make_sc_agg(True)
_sc_agg = _make_sc_agg(False)


def _tc_pre_body(x_ref, w1n_ref, w1s_ref, b1_ref, xw_ref, xs_ref):
  x = x_ref[...]
  xw_ref[...] = jnp.dot(x, w1n_ref[...], preferred_element_type=jnp.float32)
  xs_ref[...] = (jnp.dot(x, w1s_ref[...], preferred_element_type=jnp.float32)
                 + b1_ref[...])


def _tc_mid_body(xs_ref, agg_ref, deg_ref, w2n_ref, w2s_ref, b2_ref,
                 xw2_ref, h1s_ref):
  agg = agg_ref[0] + agg_ref[1]
  deg = deg_ref[0] + deg_ref[1]
  dinv = 1.0 / jnp.maximum(deg[:, 0:1], 1.0)
  h1 = jnp.maximum(xs_ref[...] + agg * dinv, 0.0)
  xw2_ref[...] = jnp.dot(h1, w2n_ref[...], preferred_element_type=jnp.float32)
  h1s_ref[...] = (jnp.dot(h1, w2s_ref[...], preferred_element_type=jnp.float32)
                  + b2_ref[...])


def _tc_fin_body(h1s_ref, agg_ref, deg_ref, out_ref):
  agg = agg_ref[0] + agg_ref[1]
  deg = deg_ref[0] + deg_ref[1]
  dinv = 1.0 / jnp.maximum(deg[:, 0:1], 1.0)
  out_ref[...] = h1s_ref[...] + agg * dinv


_row_spec = pl.BlockSpec((BN, D), lambda i: (i, 0))
_w_spec = pl.BlockSpec((D, D), lambda i: (0, 0))
_b_spec = pl.BlockSpec((1, D), lambda i: (0, 0))
_agg_spec = pl.BlockSpec((NC, BN, D), lambda i: (0, i, 0))
_deg_spec = pl.BlockSpec((NC, BN, 16), lambda i: (0, i, 0))

_tc_pre = pl.pallas_call(
    _tc_pre_body,
    grid=(N // BN,),
    in_specs=[_row_spec, _w_spec, _w_spec, _b_spec],
    out_specs=[_row_spec, _row_spec],
    out_shape=[jax.ShapeDtypeStruct((N, D), jnp.float32)] * 2,
)

_tc_mid = pl.pallas_call(
    _tc_mid_body,
    grid=(N // BN,),
    in_specs=[_row_spec, _agg_spec, _deg_spec, _w_spec, _w_spec, _b_spec],
    out_specs=[_row_spec, _row_spec],
    out_shape=[jax.ShapeDtypeStruct((N, D), jnp.float32)] * 2,
)

_tc_fin = pl.pallas_call(
    _tc_fin_body,
    grid=(N // BN,),
    in_specs=[_row_spec, _agg_spec, _deg_spec],
    out_specs=_row_spec,
    out_shape=jax.ShapeDtypeStruct((N, D), jnp.float32),
)


@jax.jit
def _run(x, edge_index, W1_self, W1_neigh, b1, W2_self, W2_neigh, b2):
  src = edge_index[0]
  dst = edge_index[1]
  zrow = jnp.zeros((ZCH, D), jnp.float32)
  zdeg = jnp.zeros((WROWS, 16), jnp.float32)
  ones_c = jnp.ones((CH, 16), jnp.float32)

  xw1, xs1 = _tc_pre(x, W1_neigh.T, W1_self.T, b1[None, :])
  aggp1, degp = _sc_agg_deg(xw1, src, dst, zrow, zdeg, ones_c)
  xw2, h1s = _tc_mid(xs1, aggp1, degp, W2_neigh.T, W2_self.T, b2[None, :])
  (aggp2,) = _sc_agg(xw2, src, dst, zrow)
  return _tc_fin(h1s, aggp2, degp)


def kernel(x, edge_index, W1_self, W1_neigh, b1, W2_self, W2_neigh, b2):
  return _run(x, edge_index, W1_self, W1_neigh, b1, W2_self, W2_neigh, b2)


# speedup vs baseline: 11.4967x; 2.1551x over previous
import functools

import jax
import jax.numpy as jnp
from jax import lax
from jax.experimental import pallas as pl
from jax.experimental.pallas import tpu as pltpu
from jax.experimental.pallas import tpu_sc as plsc

N = 10000
E = 320000
D = 128

NC = 2
NS = 16
NW = NC * NS
CH = 80
EW = E // NW
NCHUNK = EW // CH
W0 = 624
WROWS = 640
DW = 8
BN = 1000


def _make_sc_agg(with_deg):
  mesh = plsc.VectorSubcoreMesh(core_axis_name="c", subcore_axis_name="s")
  out_type = [jax.ShapeDtypeStruct((NC, N, D), jnp.float32)]
  scratch = [
      pltpu.VMEM((EW,), jnp.int32),
      pltpu.VMEM((NCHUNK, CH), jnp.int32),
      pltpu.VMEM((CH, D), jnp.float32),
      pltpu.VMEM((CH, D), jnp.float32),
      pltpu.VMEM_SHARED((N, D), jnp.float32),
      pltpu.SemaphoreType.DMA,
      pltpu.SemaphoreType.DMA,
  ]
  if with_deg:
    out_type.append(jax.ShapeDtypeStruct((NC, N, DW), jnp.float32))
    scratch += [
        pltpu.VMEM((CH, DW), jnp.float32),
        pltpu.VMEM_SHARED((N, DW), jnp.float32),
    ]

  def body(*refs):
    if with_deg:
      (xw, src, dst, zrow, zdeg, ones_h,
       agg_out, deg_out,
       sidx, didx, rowsA, rowsB, agg_sh, semA, semB,
       ones_v, deg_sh) = refs
    else:
      (xw, src, dst, zrow,
       agg_out,
       sidx, didx, rowsA, rowsB, agg_sh, semA, semB) = refs
    c = lax.axis_index("c")
    s = lax.axis_index("s")
    wid = s * NC + c
    row0 = s * W0

    pltpu.sync_copy(zrow, agg_sh.at[pl.ds(row0, WROWS)])
    if with_deg:
      pltpu.sync_copy(ones_h, ones_v)
      pltpu.sync_copy(zdeg, deg_sh.at[pl.ds(row0, WROWS)])
    pltpu.sync_copy(src.at[wid], sidx)
    pltpu.sync_copy(dst.at[wid], didx)
    plsc.subcore_barrier()

    def gather_start(j, rows, sem):
      pltpu.async_copy(xw.at[sidx.at[pl.ds(j * CH, CH)]], rows, sem)

    def gather_wait(rows, sem):
      pltpu.make_async_copy(xw.at[pl.ds(0, CH)], rows, sem).wait()

    def scatter(j, rows):
      pltpu.sync_copy(rows, agg_sh.at[didx.at[j]], add=True)
      if with_deg:
        pltpu.sync_copy(ones_v, deg_sh.at[didx.at[j]], add=True)

    gather_start(0, rowsA, semA)

    def pipe(jj, carry):
      j0 = 2 * jj
      gather_start(j0 + 1, rowsB, semB)
      gather_wait(rowsA, semA)
      scatter(j0, rowsA)
      gather_start(j0 + 2, rowsA, semA)
      gather_wait(rowsB, semB)
      scatter(j0 + 1, rowsB)
      return carry

    lax.fori_loop(0, (NCHUNK - 1) // 2, pipe, 0)
    gather_wait(rowsA, semA)
    scatter(NCHUNK - 1, rowsA)
    plsc.subcore_barrier()

    pltpu.sync_copy(agg_sh.at[pl.ds(row0, WROWS)],
                    agg_out.at[c, pl.ds(row0, WROWS)])
    if with_deg:
      pltpu.sync_copy(deg_sh.at[pl.ds(row0, WROWS)],
                      deg_out.at[c, pl.ds(row0, WROWS)])

  return pl.kernel(
      body, out_type=out_type, mesh=mesh, scratch_types=scratch,
      compiler_params=pltpu.CompilerParams(use_tc_tiling_on_sc=False))


_sc_agg_deg = _make_sc_agg(True)
_sc_agg = _make_sc_agg(False)


def _tc_pre_body(x_ref, w1n_ref, w1s_ref, b1_ref, xw_ref, xs_ref):
  x = x_ref[...]
  xw_ref[...] = jnp.dot(x, w1n_ref[...], preferred_element_type=jnp.float32)
  xs_ref[...] = (jnp.dot(x, w1s_ref[...], preferred_element_type=jnp.float32)
                 + b1_ref[...])


def _tc_mid_body(xs_ref, agg_ref, deg_ref, w2n_ref, w2s_ref, b2_ref,
                 xw2_ref, h1s_ref):
  agg = agg_ref[0] + agg_ref[1]
  deg = deg_ref[0] + deg_ref[1]
  dinv = 1.0 / jnp.maximum(deg[:, 0:1], 1.0)
  h1 = jnp.maximum(xs_ref[...] + agg * dinv, 0.0)
  xw2_ref[...] = jnp.dot(h1, w2n_ref[...], preferred_element_type=jnp.float32)
  h1s_ref[...] = (jnp.dot(h1, w2s_ref[...], preferred_element_type=jnp.float32)
                  + b2_ref[...])


def _tc_fin_body(h1s_ref, agg_ref, deg_ref, out_ref):
  agg = agg_ref[0] + agg_ref[1]
  deg = deg_ref[0] + deg_ref[1]
  dinv = 1.0 / jnp.maximum(deg[:, 0:1], 1.0)
  out_ref[...] = h1s_ref[...] + agg * dinv


_row_spec = pl.BlockSpec((BN, D), lambda i: (i, 0))
_w_spec = pl.BlockSpec((D, D), lambda i: (0, 0))
_b_spec = pl.BlockSpec((1, D), lambda i: (0, 0))
_agg_spec = pl.BlockSpec((NC, BN, D), lambda i: (0, i, 0))
_deg_spec = pl.BlockSpec((NC, BN, DW), lambda i: (0, i, 0))

_tc_pre = pl.pallas_call(
    _tc_pre_body,
    grid=(N // BN,),
    in_specs=[_row_spec, _w_spec, _w_spec, _b_spec],
    out_specs=[_row_spec, _row_spec],
    out_shape=[jax.ShapeDtypeStruct((N, D), jnp.float32)] * 2,
)

_tc_mid = pl.pallas_call(
    _tc_mid_body,
    grid=(N // BN,),
    in_specs=[_row_spec, _agg_spec, _deg_spec, _w_spec, _w_spec, _b_spec],
    out_specs=[_row_spec, _row_spec],
    out_shape=[jax.ShapeDtypeStruct((N, D), jnp.float32)] * 2,
)

_tc_fin = pl.pallas_call(
    _tc_fin_body,
    grid=(N // BN,),
    in_specs=[_row_spec, _agg_spec, _deg_spec],
    out_specs=_row_spec,
    out_shape=jax.ShapeDtypeStruct((N, D), jnp.float32),
)


@jax.jit
def _run(x, edge_index, W1_self, W1_neigh, b1, W2_self, W2_neigh, b2):
  src = edge_index[0].reshape(NW, EW)
  dst = edge_index[1].reshape(NW, NCHUNK, CH)
  zrow = jnp.zeros((WROWS, D), jnp.float32)
  zdeg = jnp.zeros((WROWS, DW), jnp.float32)
  ones_c = jnp.ones((CH, DW), jnp.float32)

  xw1, xs1 = _tc_pre(x, W1_neigh.T, W1_self.T, b1[None, :])
  aggp1, degp = _sc_agg_deg(xw1, src, dst, zrow, zdeg, ones_c)
  xw2, h1s = _tc_mid(xs1, aggp1, degp, W2_neigh.T, W2_self.T, b2[None, :])
  (aggp2,) = _sc_agg(xw2, src, dst, zrow)
  return _tc_fin(h1s, aggp2, degp)


def kernel(x, edge_index, W1_self, W1_neigh, b1, W2_self, W2_neigh, b2):
  return _run(x, edge_index, W1_self, W1_neigh, b1, W2_self, W2_neigh, b2)
